# Initial kernel scaffold; baseline (speedup 1.0000x reference)
import jax
import jax.numpy as jnp
from jax.experimental import pallas as pl


def kernel(input, embedding):
    raise NotImplementedError("write your pallas kernel here")


# speedup vs baseline: 3.9366x; 3.9366x over previous
import functools

import jax
import jax.numpy as jnp
from jax import lax
from jax.experimental import pallas as pl
from jax.experimental.pallas import tpu as pltpu
from jax.experimental.pallas import tpu_sc as plsc

MAXP = 4096
DEMB = 64
NW = 32
CHUNK = 128
FIRE = 8
LANES = 16


def kernel(input, embedding):
    S, W = input.shape
    k = min((S - 1) // 2, MAXP)
    B = S * W
    per_w = B // NW
    nt = per_w // (FIRE * CHUNK)

    inp = input.reshape(NW, nt, FIRE, CHUNK).astype(jnp.int32)
    mesh = plsc.VectorSubcoreMesh(core_axis_name="c", subcore_axis_name="s")

    @functools.partial(
        pl.kernel,
        mesh=mesh,
        out_type=jax.ShapeDtypeStruct((NW, nt, FIRE, CHUNK, DEMB), jnp.float32),
        scratch_types=[
            pltpu.VMEM((FIRE, CHUNK), jnp.int32),
            pltpu.VMEM((FIRE, CHUNK, DEMB), jnp.float32),
            pltpu.SemaphoreType.DMA,
        ],
        compiler_params=pltpu.CompilerParams(use_tc_tiling_on_sc=False),
    )
    def body(inp_hbm, emb_hbm, out_hbm, idx_v, rows_v, sem):
        nc = 2
        wid = lax.axis_index("s") * nc + lax.axis_index("c")

        def step(t, carry):
            pltpu.sync_copy(inp_hbm.at[wid, t], idx_v)

            def fix(j, c):
                for q in range(CHUNK // LANES):
                    v = idx_v[j, pl.ds(q * LANES, LANES)]
                    v = jnp.clip(v, -MAXP, MAXP) + k
                    idx_v[j, pl.ds(q * LANES, LANES)] = v
                return c

            lax.fori_loop(0, FIRE, fix, 0)

            copies = [
                pltpu.async_copy(emb_hbm.at[idx_v.at[a]], rows_v.at[a], sem)
                for a in range(FIRE)
            ]
            for c in copies:
                c.wait()
            pltpu.sync_copy(rows_v, out_hbm.at[wid, t])
            return carry

        lax.fori_loop(0, nt, step, 0)

    out = body(inp, embedding)
    return out.reshape(S, W, DEMB)


# speedup vs baseline: 3.9850x; 1.0123x over previous
import functools

import jax
import jax.numpy as jnp
from jax import lax
from jax.experimental import pallas as pl
from jax.experimental.pallas import tpu as pltpu
from jax.experimental.pallas import tpu_sc as plsc

MAXP = 4096
DEMB = 64
NW = 32
CHUNK = 128
FIRE = 4
LANES = 16


def kernel(input, embedding):
    S, W = input.shape
    k = min((S - 1) // 2, MAXP)
    B = S * W
    per_w = B // NW
    nrows = per_w // CHUNK
    nt = nrows // FIRE

    inp = input.reshape(NW, nrows, CHUNK).astype(jnp.int32)
    mesh = plsc.VectorSubcoreMesh(core_axis_name="c", subcore_axis_name="s")

    @functools.partial(
        pl.kernel,
        mesh=mesh,
        out_type=jax.ShapeDtypeStruct((NW, nt, FIRE, CHUNK, DEMB), jnp.float32),
        scratch_types=[
            pltpu.VMEM((nrows, CHUNK), jnp.int32),
            pltpu.VMEM((2, FIRE, CHUNK, DEMB), jnp.float32),
            pltpu.SemaphoreType.DMA,
            pltpu.SemaphoreType.DMA,
        ],
        compiler_params=pltpu.CompilerParams(use_tc_tiling_on_sc=False),
    )
    def body(inp_hbm, emb_hbm, out_hbm, idx_v, rows_v, gsem, wsem):
        nc = 2
        wid = lax.axis_index("s") * nc + lax.axis_index("c")

        pltpu.sync_copy(inp_hbm.at[wid], idx_v)

        def transform(t):
            def fix(r, c):
                for q in range(CHUNK // LANES):
                    v = idx_v[r, pl.ds(q * LANES, LANES)]
                    v = jnp.clip(v, -MAXP, MAXP) + k
                    idx_v[r, pl.ds(q * LANES, LANES)] = v
                return c

            lax.fori_loop(t * FIRE, (t + 1) * FIRE, fix, 0)

        def fire_gathers(t, p):
            for a in range(FIRE):
                pltpu.async_copy(
                    emb_hbm.at[idx_v.at[t * FIRE + a]], rows_v.at[p, a], gsem
                )

        def drain_gathers():
            for a in range(FIRE):
                pltpu.make_async_copy(
                    emb_hbm.at[idx_v.at[0]], rows_v.at[0, a], gsem
                ).wait()

        transform(0)
        fire_gathers(0, 0)

        def step(t, carry):
            p = lax.rem(t, 2)

            @pl.when(t + 1 < nt)
            def _():
                transform(t + 1)

            drain_gathers()

            @pl.when(t >= 1)
            def _():
                pltpu.make_async_copy(rows_v.at[0], out_hbm.at[wid, 0], wsem).wait()

            @pl.when(t + 1 < nt)
            def _():
                fire_gathers(t + 1, 1 - p)

            pltpu.async_copy(rows_v.at[p], out_hbm.at[wid, t], wsem)
            return carry

        lax.fori_loop(0, nt, step, 0)
        pltpu.make_async_copy(rows_v.at[0], out_hbm.at[wid, 0], wsem).wait()

    out = body(inp, embedding)
    return out.reshape(S, W, DEMB)
